# baseline (speedup 1.0000x reference)
import functools

import jax
import jax.numpy as jnp
from jax import lax
from jax.experimental import pallas as pl
from jax.experimental.pallas import tpu as pltpu
from jax.experimental.pallas import tpu_sc as plsc

B = 16384
N_COLORS = 32
DIM_COLOR = 16
OV = 34
L_OR = 20
DIM_ORACLE = 32
DIM_LATENT = 64
N_Q = 1000000


_NC = 2
_NS = 16
_NW = _NC * _NS
_BPW = B // _NW
_CH = 128
_NCH = _BPW // _CH


_NFLAT = _BPW * DIM_LATENT


def _latent_body(idx_hbm, wt_hbm, out_hbm, idx_v, idxs, stage, sem):
    wid = lax.axis_index("s") * _NC + lax.axis_index("c")
    base = wid * _BPW
    pltpu.sync_copy(idx_hbm.at[wid], idx_v)

    def build(g, carry):
        vec = idx_v[pl.ds(g * 16, 16)]
        for d in range(DIM_LATENT):
            idxs[pl.ds(d * _BPW + g * 16, 16)] = vec + (N_Q + 1) * d
        return carry

    lax.fori_loop(0, _BPW // 16, build, 0)

    def gather(it, carry):
        copies = []
        for k in range(8):
            off = it * 1024 + k * 128
            copies.append(
                pltpu.async_copy(
                    wt_hbm.at[idxs.at[pl.ds(off, 128)]],
                    stage.at[pl.ds(off, 128)],
                    sem,
                )
            )
        for c in copies:
            c.wait()
        return carry

    lax.fori_loop(0, _NFLAT // 1024, gather, 0)

    out_copies = [
        pltpu.async_copy(
            stage.at[pl.ds(d * _BPW, _BPW)],
            out_hbm.at[d, pl.ds(base, _BPW)],
            sem,
        )
        for d in range(DIM_LATENT)
    ]
    for c in out_copies:
        c.wait()


@functools.cache
def _latent_call():
    return functools.partial(
        pl.kernel,
        mesh=plsc.VectorSubcoreMesh(core_axis_name="c", subcore_axis_name="s"),
        out_type=jax.ShapeDtypeStruct((DIM_LATENT, B), jnp.float32),
        scratch_types=[
            pltpu.VMEM((_BPW,), jnp.int32),
            pltpu.VMEM((_NFLAT,), jnp.int32),
            pltpu.VMEM((_NFLAT,), jnp.float32),
            pltpu.SemaphoreType.DMA,
        ],
        compiler_params=pltpu.CompilerParams(use_tc_tiling_on_sc=False),
    )(_latent_body)


_BLK = 1024
_GRID = B // _BLK


_DOUT = DIM_COLOR + DIM_ORACLE + DIM_LATENT


def _co_body(qc_ref, qo_ref, lat_ref, wc_ref, wo_ref, out_ref):
    qc = qc_ref[0]
    qo = qo_ref[0]
    oh_c = (qc == lax.broadcasted_iota(jnp.int32, (_BLK, N_COLORS), 1)).astype(
        jnp.float32
    )
    color = jnp.dot(oh_c, wc_ref[:, :], preferred_element_type=jnp.float32)
    cnt = jnp.zeros((_BLK, OV), jnp.float32)
    for l in range(L_OR):
        cnt = cnt + (
            qo[:, l : l + 1] == lax.broadcasted_iota(jnp.int32, (_BLK, OV), 1)
        ).astype(jnp.float32)
    oracle = jnp.dot(cnt, wo_ref[:, :], preferred_element_type=jnp.float32) * (
        1.0 / L_OR
    )
    out_ref[0] = jnp.concatenate([color, oracle, lat_ref[0]], axis=1)


def _make_co_call(interpret=False):
    return pl.pallas_call(
        _co_body,
        grid=(_GRID,),
        in_specs=[
            pl.BlockSpec((1, _BLK, 1), lambda i: (i, 0, 0)),
            pl.BlockSpec((1, _BLK, L_OR), lambda i: (i, 0, 0)),
            pl.BlockSpec((1, _BLK, DIM_LATENT), lambda i: (i, 0, 0)),
            pl.BlockSpec((N_COLORS, DIM_COLOR), lambda i: (0, 0)),
            pl.BlockSpec((OV, DIM_ORACLE), lambda i: (0, 0)),
        ],
        out_specs=pl.BlockSpec((1, _BLK, _DOUT), lambda i: (i, 0, 0)),
        out_shape=jax.ShapeDtypeStruct((_GRID, _BLK, _DOUT), jnp.float32),
        interpret=interpret,
    )


_co_call = _make_co_call()


def kernel(q_color, q_oracle, q_emb_input, W_color, W_oracle, W_latent):
    qc3 = q_color.astype(jnp.int32).reshape(_GRID, _BLK, 1)
    qo3 = q_oracle.astype(jnp.int32).reshape(_GRID, _BLK, L_OR)
    idx = q_emb_input.astype(jnp.int32).reshape(_NW, _BPW)
    lat_t = _latent_call()(idx, W_latent.T.reshape(-1))
    latent = lat_t.T.reshape(_GRID, _BLK, DIM_LATENT)
    out = _co_call(qc3, qo3, latent, W_color, W_oracle)
    return out.reshape(B, _DOUT)


# speedup vs baseline: 3.6754x; 3.6754x over previous
import functools

import jax
import jax.numpy as jnp
from jax import lax
from jax.experimental import pallas as pl
from jax.experimental.pallas import tpu as pltpu
from jax.experimental.pallas import tpu_sc as plsc

B = 16384
N_COLORS = 32
DIM_COLOR = 16
OV = 34
L_OR = 20
DIM_ORACLE = 32
DIM_LATENT = 64
N_Q = 1000000
PAIR_ROWS = (N_Q + 2) // 2


_NC = 2
_NS = 16
_NW = _NC * _NS
_BPW = B // _NW
_CH = 128
_NCH = _BPW // _CH


def _latent_body(idx_hbm, table_hbm, out_hbm, idx_v, rows_v, sem):
    wid = lax.axis_index("s") * _NC + lax.axis_index("c")
    base = wid * _BPW
    pltpu.sync_copy(idx_hbm.at[wid], idx_v)
    copies = [
        pltpu.async_copy(
            table_hbm.at[idx_v.at[j]],
            rows_v.at[pl.ds(j * _CH, _CH)],
            sem,
        )
        for j in range(_NCH)
    ]
    for c in copies:
        c.wait()
    pltpu.sync_copy(rows_v, out_hbm.at[pl.ds(base, _BPW)])


@functools.cache
def _latent_call():
    return functools.partial(
        pl.kernel,
        mesh=plsc.VectorSubcoreMesh(core_axis_name="c", subcore_axis_name="s"),
        out_type=jax.ShapeDtypeStruct((B, 2 * DIM_LATENT), jnp.bfloat16),
        scratch_types=[
            pltpu.VMEM((_NCH, _CH), jnp.int32),
            pltpu.VMEM((_BPW, 2 * DIM_LATENT), jnp.bfloat16),
            pltpu.SemaphoreType.DMA,
        ],
        compiler_params=pltpu.CompilerParams(use_tc_tiling_on_sc=False),
    )(_latent_body)


_BLK = 1024
_GRID = B // _BLK
_DOUT = DIM_COLOR + DIM_ORACLE + DIM_LATENT


def _co_body(qc_ref, qo_ref, par_ref, lat_ref, wc_ref, wo_ref, out_ref):
    qc = qc_ref[0]
    qo = qo_ref[0]
    oh_c = (qc == lax.broadcasted_iota(jnp.int32, (_BLK, N_COLORS), 1)).astype(
        jnp.float32
    )
    color = jnp.dot(oh_c, wc_ref[:, :], preferred_element_type=jnp.float32)
    cnt = jnp.zeros((_BLK, OV), jnp.float32)
    for l in range(L_OR):
        cnt = cnt + (
            qo[:, l : l + 1] == lax.broadcasted_iota(jnp.int32, (_BLK, OV), 1)
        ).astype(jnp.float32)
    oracle = jnp.dot(cnt, wo_ref[:, :], preferred_element_type=jnp.float32) * (
        1.0 / L_OR
    )
    pair = lat_ref[0].astype(jnp.float32)
    odd = par_ref[0] == 1
    latent = jnp.where(odd, pair[:, DIM_LATENT:], pair[:, :DIM_LATENT])
    out_ref[0] = jnp.concatenate([color, oracle, latent], axis=1)


def _make_co_call(interpret=False):
    return pl.pallas_call(
        _co_body,
        grid=(_GRID,),
        in_specs=[
            pl.BlockSpec((1, _BLK, 1), lambda i: (i, 0, 0)),
            pl.BlockSpec((1, _BLK, L_OR), lambda i: (i, 0, 0)),
            pl.BlockSpec((1, _BLK, 1), lambda i: (i, 0, 0)),
            pl.BlockSpec((1, _BLK, 2 * DIM_LATENT), lambda i: (i, 0, 0)),
            pl.BlockSpec((N_COLORS, DIM_COLOR), lambda i: (0, 0)),
            pl.BlockSpec((OV, DIM_ORACLE), lambda i: (0, 0)),
        ],
        out_specs=pl.BlockSpec((1, _BLK, _DOUT), lambda i: (i, 0, 0)),
        out_shape=jax.ShapeDtypeStruct((_GRID, _BLK, _DOUT), jnp.float32),
        interpret=interpret,
    )


_co_call = _make_co_call()


def kernel(q_color, q_oracle, q_emb_input, W_color, W_oracle, W_latent):
    qe = q_emb_input.astype(jnp.int32)
    table16 = (
        jnp.pad(W_latent, ((0, 1), (0, 0)))
        .astype(jnp.bfloat16)
        .reshape(PAIR_ROWS, 2 * DIM_LATENT)
    )
    pidx = (qe // 2).reshape(_NW, _NCH, _CH)
    lat_pairs = _latent_call()(pidx, table16)
    qc3 = q_color.astype(jnp.int32).reshape(_GRID, _BLK, 1)
    qo3 = q_oracle.astype(jnp.int32).reshape(_GRID, _BLK, L_OR)
    par3 = (qe % 2).reshape(_GRID, _BLK, 1)
    lat3 = lat_pairs.reshape(_GRID, _BLK, 2 * DIM_LATENT)
    out = _co_call(qc3, qo3, par3, lat3, W_color, W_oracle)
    return out.reshape(B, _DOUT)


# speedup vs baseline: 7.2179x; 1.9638x over previous
import functools

import jax
import jax.numpy as jnp
from jax import lax
from jax.experimental import pallas as pl
from jax.experimental.pallas import tpu as pltpu
from jax.experimental.pallas import tpu_sc as plsc

B = 16384
N_COLORS = 32
DIM_COLOR = 16
OV = 34
L_OR = 20
DIM_ORACLE = 32
DIM_LATENT = 64
N_Q = 1000000


_NC = 2
_NS = 16
_NW = _NC * _NS
_BPW = B // _NW
_CH = 128
_NCH = _BPW // _CH


def _latent_body(idx_hbm, table_hbm, out_hbm, idx_v, rows_v, sem):
    wid = lax.axis_index("s") * _NC + lax.axis_index("c")
    base = wid * _BPW
    pltpu.sync_copy(idx_hbm.at[wid], idx_v)
    copies = [
        pltpu.async_copy(
            table_hbm.at[idx_v.at[j]],
            rows_v.at[pl.ds(j * _CH, _CH)],
            sem,
        )
        for j in range(_NCH)
    ]
    for c in copies:
        c.wait()
    pltpu.sync_copy(rows_v, out_hbm.at[pl.ds(base, _BPW)])


@functools.cache
def _latent_call():
    return functools.partial(
        pl.kernel,
        mesh=plsc.VectorSubcoreMesh(core_axis_name="c", subcore_axis_name="s"),
        out_type=jax.ShapeDtypeStruct((B, DIM_LATENT), jnp.float32),
        scratch_types=[
            pltpu.VMEM((_NCH, _CH), jnp.int32),
            pltpu.VMEM((_BPW, DIM_LATENT), jnp.float32),
            pltpu.SemaphoreType.DMA,
        ],
        compiler_params=pltpu.CompilerParams(use_tc_tiling_on_sc=False),
    )(_latent_body)


_BLK = 1024
_GRID = B // _BLK
_DOUT = DIM_COLOR + DIM_ORACLE + DIM_LATENT


def _co_body(qc_ref, qo_ref, wc_ref, wo_ref, out_ref):
    qc = qc_ref[0]
    qo = qo_ref[0]
    oh_c = (qc == lax.broadcasted_iota(jnp.int32, (_BLK, N_COLORS), 1)).astype(
        jnp.float32
    )
    color = jnp.dot(oh_c, wc_ref[:, :], preferred_element_type=jnp.float32)
    cnt = jnp.zeros((_BLK, OV), jnp.float32)
    for l in range(L_OR):
        cnt = cnt + (
            qo[:, l : l + 1] == lax.broadcasted_iota(jnp.int32, (_BLK, OV), 1)
        ).astype(jnp.float32)
    oracle = jnp.dot(cnt, wo_ref[:, :], preferred_element_type=jnp.float32) * (
        1.0 / L_OR
    )
    out_ref[0] = jnp.concatenate([color, oracle], axis=1)


def _make_co_call(interpret=False):
    return pl.pallas_call(
        _co_body,
        grid=(_GRID,),
        in_specs=[
            pl.BlockSpec((1, _BLK, 1), lambda i: (i, 0, 0)),
            pl.BlockSpec((1, _BLK, L_OR), lambda i: (i, 0, 0)),
            pl.BlockSpec((N_COLORS, DIM_COLOR), lambda i: (0, 0)),
            pl.BlockSpec((OV, DIM_ORACLE), lambda i: (0, 0)),
        ],
        out_specs=pl.BlockSpec(
            (1, _BLK, DIM_COLOR + DIM_ORACLE), lambda i: (i, 0, 0)
        ),
        out_shape=jax.ShapeDtypeStruct(
            (_GRID, _BLK, DIM_COLOR + DIM_ORACLE), jnp.float32
        ),
        interpret=interpret,
    )


_co_call = _make_co_call()


def _asm_body(co_ref, lat_ref, out_ref):
    out_ref[0] = jnp.concatenate([co_ref[0], lat_ref[0]], axis=1)


_asm_call = pl.pallas_call(
    _asm_body,
    grid=(_GRID,),
    in_specs=[
        pl.BlockSpec((1, _BLK, DIM_COLOR + DIM_ORACLE), lambda i: (i, 0, 0)),
        pl.BlockSpec((1, _BLK, DIM_LATENT), lambda i: (i, 0, 0)),
    ],
    out_specs=pl.BlockSpec((1, _BLK, _DOUT), lambda i: (i, 0, 0)),
    out_shape=jax.ShapeDtypeStruct((_GRID, _BLK, _DOUT), jnp.float32),
)


def kernel(q_color, q_oracle, q_emb_input, W_color, W_oracle, W_latent):
    qc3 = q_color.astype(jnp.int32).reshape(_GRID, _BLK, 1)
    qo3 = q_oracle.astype(jnp.int32).reshape(_GRID, _BLK, L_OR)
    idx = q_emb_input.astype(jnp.int32).reshape(_NW, _NCH, _CH)
    latent = _latent_call()(idx, W_latent).reshape(_GRID, _BLK, DIM_LATENT)
    co = _co_call(qc3, qo3, W_color, W_oracle)
    out = _asm_call(co, latent)
    return out.reshape(B, _DOUT)


# speedup vs baseline: 7.9924x; 1.1073x over previous
import functools

import jax
import jax.numpy as jnp
from jax import lax
from jax.experimental import pallas as pl
from jax.experimental.pallas import tpu as pltpu
from jax.experimental.pallas import tpu_sc as plsc

B = 16384
N_COLORS = 32
DIM_COLOR = 16
OV = 34
L_OR = 20
DIM_ORACLE = 32
DIM_LATENT = 64
N_Q = 1000000


_NC = 2
_NS = 16
_NW = _NC * _NS
_BPW = B // _NW
_CH = 128
_NCH = _BPW // _CH


def _latent_body(idx_hbm, table_hbm, out_hbm, idx_v, rows_v, sem):
    wid = lax.axis_index("s") * _NC + lax.axis_index("c")
    base = wid * _BPW
    pltpu.sync_copy(idx_hbm.at[wid], idx_v)
    copies = [
        pltpu.async_copy(
            table_hbm.at[idx_v.at[j]],
            rows_v.at[pl.ds(j * _CH, _CH)],
            sem,
        )
        for j in range(_NCH)
    ]
    for c in copies:
        c.wait()
    pltpu.sync_copy(rows_v, out_hbm.at[pl.ds(base, _BPW)])


@functools.cache
def _latent_call():
    return functools.partial(
        pl.kernel,
        mesh=plsc.VectorSubcoreMesh(core_axis_name="c", subcore_axis_name="s"),
        out_type=jax.ShapeDtypeStruct((B, DIM_LATENT), jnp.float32),
        scratch_types=[
            pltpu.VMEM((_NCH, _CH), jnp.int32),
            pltpu.VMEM((_BPW, DIM_LATENT), jnp.float32),
            pltpu.SemaphoreType.DMA,
        ],
        compiler_params=pltpu.CompilerParams(use_tc_tiling_on_sc=False),
    )(_latent_body)


_BLK = 1024
_GRID = B // _BLK
_DOUT = DIM_COLOR + DIM_ORACLE + DIM_LATENT


def _co_body(qc_ref, qo_ref, lat_ref, wct_ref, wot_ref, out_ref):
    qc = qc_ref[:, :]
    qo = qo_ref[:, :]
    oh_ct = (
        qc == lax.broadcasted_iota(jnp.int32, (N_COLORS, _BLK), 0)
    ).astype(jnp.float32)
    color_t = jnp.dot(wct_ref[:, :], oh_ct, preferred_element_type=jnp.float32)
    cnt_t = jnp.zeros((OV, _BLK), jnp.float32)
    for l in range(L_OR):
        cnt_t = cnt_t + (
            qo[l : l + 1, :] == lax.broadcasted_iota(jnp.int32, (OV, _BLK), 0)
        ).astype(jnp.float32)
    oracle_t = jnp.dot(
        wot_ref[:, :], cnt_t, preferred_element_type=jnp.float32
    ) * (1.0 / L_OR)
    out_ref[:, :] = jnp.concatenate([color_t, oracle_t, lat_ref[:, :]], axis=0)


def _make_co_call(interpret=False):
    return pl.pallas_call(
        _co_body,
        grid=(_GRID,),
        in_specs=[
            pl.BlockSpec((1, _BLK), lambda i: (0, i)),
            pl.BlockSpec((L_OR, _BLK), lambda i: (0, i)),
            pl.BlockSpec((DIM_LATENT, _BLK), lambda i: (0, i)),
            pl.BlockSpec((DIM_COLOR, N_COLORS), lambda i: (0, 0)),
            pl.BlockSpec((DIM_ORACLE, OV), lambda i: (0, 0)),
        ],
        out_specs=pl.BlockSpec((_DOUT, _BLK), lambda i: (0, i)),
        out_shape=jax.ShapeDtypeStruct((_DOUT, B), jnp.float32),
        interpret=interpret,
    )


_co_call = _make_co_call()


def kernel(q_color, q_oracle, q_emb_input, W_color, W_oracle, W_latent):
    qc2 = q_color.astype(jnp.int32).reshape(1, B)
    qo_t = q_oracle.astype(jnp.int32).T
    idx = q_emb_input.astype(jnp.int32).reshape(_NW, _NCH, _CH)
    lat_t = _latent_call()(idx, W_latent).T
    out_t = _co_call(qc2, qo_t, lat_t, W_color.T, W_oracle.T)
    return out_t.T
